# baseline (speedup 1.0000x reference)
import functools
import math

import jax
import jax.numpy as jnp
from jax.experimental import pallas as pl

DIM = 256
HEADS = 4
TOPK = 40
MLP = 2
EPS = 1e-5
RS = 2

HD = DIM // HEADS
NEG = -1e30


def _conv1d(x, w, b=None, padding=0, groups=1):
    out = jax.lax.conv_general_dilated(
        x, w, (1,), [(padding, padding)],
        dimension_numbers=('NCH', 'OIH', 'NCH'),
        feature_group_count=groups)
    if b is not None:
        out = out + b[None, :, None]
    return out


def _bn(x, g, b):
    return x / jnp.sqrt(1.0 + EPS) * g[None, :, None] + b[None, :, None]


def _attn_kernel(q_ref, k_ref, v_ref, idx_ref, o_ref, *, nr, rb, scale):
    idxb = idx_ref[0]
    cols = jax.lax.broadcasted_iota(jnp.int32, (rb, nr), 1)
    mask = jnp.zeros((rb, nr), jnp.bool_)
    for j in range(TOPK):
        mask = jnp.logical_or(mask, cols == idxb[:, j:j + 1])

    k0 = k_ref[0, 0, 0]
    k1 = k_ref[0, 0, 1]
    v0 = v_ref[0, 0, 0]
    v1 = v_ref[0, 0, 1]
    for i in range(RS):
        q = q_ref[0, 0, i]
        s0 = jax.lax.dot_general(q, k0, (((1,), (1,)), ((), ())),
                                 preferred_element_type=jnp.float32)
        s1 = jax.lax.dot_general(q, k1, (((1,), (1,)), ((), ())),
                                 preferred_element_type=jnp.float32)
        s0 = jnp.where(mask, s0 * scale, NEG)
        s1 = jnp.where(mask, s1 * scale, NEG)
        mx = jnp.maximum(jnp.max(s0, axis=1, keepdims=True),
                         jnp.max(s1, axis=1, keepdims=True))
        p0 = jnp.exp(s0 - mx)
        p1 = jnp.exp(s1 - mx)
        den = (jnp.sum(p0, axis=1, keepdims=True)
               + jnp.sum(p1, axis=1, keepdims=True))
        o = (jax.lax.dot_general(p0, v0, (((1,), (0,)), ((), ())),
                                 preferred_element_type=jnp.float32)
             + jax.lax.dot_general(p1, v1, (((1,), (0,)), ((), ())),
                                   preferred_element_type=jnp.float32))
        o_ref[0, 0, i] = o / den


def _masked_attention(q, k, v, idx, nr):
    Bb = q.shape[0]
    rb = 256
    grid = (Bb, HEADS, nr // rb)
    scale = DIM ** (-0.5)
    kern = functools.partial(_attn_kernel, nr=nr, rb=rb, scale=scale)
    return pl.pallas_call(
        kern,
        grid=grid,
        in_specs=[
            pl.BlockSpec((1, 1, RS, rb, HD), lambda b, h, r: (b, h, 0, r, 0)),
            pl.BlockSpec((1, 1, RS, nr, HD), lambda b, h, r: (b, h, 0, 0, 0)),
            pl.BlockSpec((1, 1, RS, nr, HD), lambda b, h, r: (b, h, 0, 0, 0)),
            pl.BlockSpec((1, rb, TOPK), lambda b, h, r: (b, r, 0)),
        ],
        out_specs=pl.BlockSpec((1, 1, RS, rb, HD),
                               lambda b, h, r: (b, h, 0, r, 0)),
        out_shape=jax.ShapeDtypeStruct((Bb, HEADS, RS, nr, HD), jnp.float32),
    )(q, k, v, idx)


def _to_pos_regions(t, nr):
    Bb = t.shape[0]
    t = t.reshape(Bb, HEADS, HD, nr, RS)
    return t.transpose(0, 1, 4, 3, 2)


def _from_pos_regions(t):
    Bb = t.shape[0]
    t = t.transpose(0, 1, 4, 3, 2)
    return t.reshape(Bb, DIM, -1)


def _attention(x, p):
    Bb, C, Tt = x.shape
    nr = Tt // RS
    q = _bn(_conv1d(x, p['q_w'], padding=1), p['q_g'], p['q_b'])
    k = _bn(_conv1d(x, p['k_w'], padding=1), p['k_g'], p['k_b'])
    v = _conv1d(x, p['v_w'])

    q_r = q.reshape(Bb, C, nr, RS).mean(-1)
    k_r = k.reshape(Bb, C, nr, RS).mean(-1)
    a_r = jnp.einsum('bcr,bcs->brs', q_r, k_r)
    _, idx = jax.lax.top_k(a_r, TOPK)

    qp = _to_pos_regions(q, nr)
    kp = _to_pos_regions(k, nr)
    vp = _to_pos_regions(v, nr)
    out = _masked_attention(qp, kp, vp, idx, nr)
    out = _from_pos_regions(out)

    out = out + _conv1d(v, p['lepe_w'], p['lepe_b'], padding=1, groups=C)
    out = _conv1d(out, p['out_w'], p['out_b'])
    return out


def _mm(a, b):
    return jax.lax.dot_general(a, b, (((1,), (0,)), ((), ())),
                               preferred_element_type=jnp.float32)


def _mmb(a, b):
    return jax.lax.dot_general(a.astype(jnp.bfloat16), b.astype(jnp.bfloat16),
                               (((1,), (0,)), ((), ())),
                               preferred_element_type=jnp.float32)


def _ffn_kernel(x_ref, w1_ref, s1_ref, b1_ref, w2_ref, s2_ref, b2_ref,
                dr_ref, di_ref, rb_ref, ib_ref, c_ref, s_ref, tw_ref,
                o_ref, *, T):
    n = T // 2
    inv = 1.0 / math.sqrt(float(T))
    h = _mmb(x_ref[0], w1_ref[...])
    h = jax.nn.relu(h * s1_ref[...] + b1_ref[...])
    hr = h.reshape(n, 2, h.shape[-1])
    e = hr[:, 0, :]
    o = hr[:, 1, :]
    C = c_ref[...]
    S = s_ref[...]
    er = _mmb(C, e)
    ei = -_mmb(S, e)
    orr = _mmb(C, o)
    oi = -_mmb(S, o)
    ck = tw_ref[:, 0:1]
    sk = tw_ref[:, 1:2]
    tor = ck * orr + sk * oi
    toi = ck * oi - sk * orr
    x1r = (er + tor) * inv
    x1i = (ei + toi) * inv
    x2r = (er - tor) * inv
    x2i = (ei - toi) * inv
    dr = dr_ref[...]
    di = di_ref[...]
    rb = rb_ref[...]
    ib = ib_ref[...]

    def freq_nl(xr, xi):
        yr = jax.nn.relu(xr * dr - xi * di + rb)
        yi = jax.nn.relu(xi * dr + xr * di + ib)
        return yr, yi

    x1r, x1i = freq_nl(x1r, x1i)
    x2r, x2i = freq_nl(x2r, x2i)
    u0r = x1r + x2r
    u0i = x1i + x2i
    d1r = x1r - x2r
    d1i = x1i - x2i
    v1r = d1r * ck - d1i * sk
    v1i = d1i * ck + d1r * sk
    ye = (_mmb(C, u0r) - _mmb(S, u0i)) * inv
    yo = (_mmb(C, v1r) - _mmb(S, v1i)) * inv
    o_ref[0, 0] = _mmb(ye, w2_ref[...]) * s2_ref[...] + b2_ref[...]
    o_ref[0, 1] = _mmb(yo, w2_ref[...]) * s2_ref[...] + b2_ref[...]


def _ffn(x, p):
    Bb, C, Tt = x.shape
    n = Tt // 2
    dh = C * MLP
    xt = x.transpose(0, 2, 1)
    w1 = p['fc1_w'][:, :, 0].T
    rs1 = 1.0 / jnp.sqrt(1.0 + EPS)
    s1 = (p['fc1_g'] * rs1)[None, :]
    b1 = p['fc1_b'][None, :]
    w2 = p['fc2_w'][:, :, 0].T
    s2 = (p['fc2_g'] * rs1)[None, :]
    b2 = p['fc2_b'][None, :]
    dr = jnp.diagonal(p['r'])[None, :]
    di = jnp.diagonal(p['i'])[None, :]
    rb = p['rb'][None, :]
    ib = p['ib'][None, :]
    kk = jnp.arange(n, dtype=jnp.float32)
    nnm = kk[:, None] * kk[None, :] * (2.0 * jnp.pi / n)
    Cm = jnp.cos(nnm)
    Sm = jnp.sin(nnm)
    ang = jnp.pi * kk / n
    tw = jnp.stack([jnp.cos(ang), jnp.sin(ang)], axis=1)

    kern = functools.partial(_ffn_kernel, T=Tt)
    full = lambda shape: pl.BlockSpec(shape, lambda b: tuple(0 for _ in shape))
    out = pl.pallas_call(
        kern,
        grid=(Bb,),
        in_specs=[
            pl.BlockSpec((1, Tt, C), lambda b: (b, 0, 0)),
            full((C, dh)), full((1, dh)), full((1, dh)),
            full((dh, C)), full((1, C)), full((1, C)),
            full((1, dh)), full((1, dh)), full((1, dh)), full((1, dh)),
            full((n, n)), full((n, n)), full((n, 2)),
        ],
        out_specs=pl.BlockSpec((1, 2, n, C), lambda b: (b, 0, 0, 0)),
        out_shape=jax.ShapeDtypeStruct((Bb, 2, n, C), jnp.float32),
    )(xt, w1, s1, b1, w2, s2, b2, dr, di, rb, ib, Cm, Sm, tw)
    return out.transpose(0, 3, 2, 1).reshape(Bb, C, Tt)


def kernel(x, params):
    x = x + _attention(_bn(x, params['n1_g'], params['n1_b']), params)
    x = x + _ffn(_bn(x, params['n2_g'], params['n2_b']), params)
    return x


# speedup vs baseline: 1.6815x; 1.6815x over previous
import functools
import math

import jax
import jax.numpy as jnp
from jax.experimental import pallas as pl

DIM = 256
HEADS = 4
TOPK = 40
MLP = 2
EPS = 1e-5
RS = 2

HD = DIM // HEADS
NEG = -1e30


def _mm(a, b, dims=((1,), (0,))):
    return jax.lax.dot_general(a, b, (dims, ((), ())),
                               preferred_element_type=jnp.float32)


def _shift_down(a):
    return jnp.concatenate([jnp.zeros((1, a.shape[1]), a.dtype), a[:-1]], 0)


def _shift_up(a):
    return jnp.concatenate([a[1:], jnp.zeros((1, a.shape[1]), a.dtype)], 0)


def _qkv_kernel(x_ref, wq_ref, wk_ref, wv_ref, sq_ref, bq_ref, sk_ref,
                bk_ref, qp_ref, kp_ref, vp_ref, qr_ref, kr_ref, *, C):
    xe = x_ref[0, 0]
    xo = x_ref[0, 1]
    xo_m = _shift_down(xo)
    xe_p = _shift_up(xe)

    def conv3(w_ref):
        w0 = w_ref[0:C]
        w1 = w_ref[C:2 * C]
        w2 = w_ref[2 * C:3 * C]
        y_e = _mm(xo_m, w0) + _mm(xe, w1) + _mm(xo, w2)
        y_o = _mm(xe, w0) + _mm(xo, w1) + _mm(xe_p, w2)
        return y_e, y_o

    q_e, q_o = conv3(wq_ref)
    q_e = q_e * sq_ref[...] + bq_ref[...]
    q_o = q_o * sq_ref[...] + bq_ref[...]
    k_e, k_o = conv3(wk_ref)
    k_e = k_e * sk_ref[...] + bk_ref[...]
    k_o = k_o * sk_ref[...] + bk_ref[...]
    wv = wv_ref[...]
    v_e = _mm(xe, wv)
    v_o = _mm(xo, wv)
    qp_ref[0, 0] = q_e
    qp_ref[0, 1] = q_o
    kp_ref[0, 0] = k_e
    kp_ref[0, 1] = k_o
    vp_ref[0, 0] = v_e
    vp_ref[0, 1] = v_o
    qr_ref[0] = (q_e + q_o) * 0.5
    kr_ref[0] = (k_e + k_o) * 0.5


def _routing_kernel(qr_ref, kr_ref, ar_ref):
    ar_ref[0] = _mm(qr_ref[0], kr_ref[0], dims=((1,), (1,)))


def _attn_kernel(q_ref, k_ref, v_ref, ar_ref, t_ref, x_ref, wo_ref, ob_ref,
                 lw_ref, lb_ref, o_ref, *, rb, n, scale):
    qb = pl.program_id(1)
    mask = ar_ref[0] >= t_ref[0]
    ve = v_ref[0, 0]
    vo = v_ref[0, 1]
    outs = []
    for i in range(RS):
        q_i = q_ref[0, i]
        heads = []
        for h in range(HEADS):
            sl = slice(h * HD, (h + 1) * HD)
            qh = q_i[:, sl]
            s0 = _mm(qh, k_ref[0, 0][:, sl], dims=((1,), (1,)))
            s1 = _mm(qh, k_ref[0, 1][:, sl], dims=((1,), (1,)))
            s0 = jnp.where(mask, s0 * scale, NEG)
            s1 = jnp.where(mask, s1 * scale, NEG)
            mx = jnp.maximum(jnp.max(s0, 1, keepdims=True),
                             jnp.max(s1, 1, keepdims=True))
            p0 = jnp.exp(s0 - mx)
            p1 = jnp.exp(s1 - mx)
            den = (jnp.sum(p0, 1, keepdims=True)
                   + jnp.sum(p1, 1, keepdims=True))
            oh = _mm(p0, ve[:, sl]) + _mm(p1, vo[:, sl])
            heads.append(oh / den)
        outs.append(jnp.concatenate(heads, axis=1))

    r0 = qb * rb
    nq = pl.num_programs(1)
    ve_b = v_ref[0, 0, pl.ds(r0, rb), :]
    vo_b = v_ref[0, 1, pl.ds(r0, rb), :]
    prev = v_ref[0, 1, pl.ds(jnp.maximum(r0 - 1, 0), 1), :]
    prev = jnp.where(qb > 0, prev, 0.0)
    nxt = v_ref[0, 0, pl.ds(jnp.minimum(r0 + rb, n - 1), 1), :]
    nxt = jnp.where(qb < nq - 1, nxt, 0.0)
    vom_b = jnp.concatenate([prev, vo_b[:-1]], 0)
    vep_b = jnp.concatenate([ve_b[1:], nxt], 0)
    lw0 = lw_ref[0:1]
    lw1 = lw_ref[1:2]
    lw2 = lw_ref[2:3]
    lb = lb_ref[...]
    lepe_e = lw0 * vom_b + lw1 * ve_b + lw2 * vo_b + lb
    lepe_o = lw0 * ve_b + lw1 * vo_b + lw2 * vep_b + lb
    wo = wo_ref[...]
    ob = ob_ref[...]
    o_ref[0, 0] = x_ref[0, 0] + _mm(outs[0] + lepe_e, wo) + ob
    o_ref[0, 1] = x_ref[0, 1] + _mm(outs[1] + lepe_o, wo) + ob


def _ffn_kernel(x_ref, w1_ref, b1_ref, w2_ref, b2_ref, dr_ref, di_ref,
                rb_ref, ib_ref, c_ref, s_ref, tw_ref, o_ref, *, T):
    n = T // 2
    inv = 1.0 / math.sqrt(float(T))
    e = jax.nn.relu(_mm(x_ref[0, 0], w1_ref[...]) + b1_ref[...])
    o = jax.nn.relu(_mm(x_ref[0, 1], w1_ref[...]) + b1_ref[...])
    C = c_ref[...]
    S = s_ref[...]
    er = _mm(C, e)
    ei = -_mm(S, e)
    orr = _mm(C, o)
    oi = -_mm(S, o)
    ck = tw_ref[:, 0:1]
    sk = tw_ref[:, 1:2]
    tor = ck * orr + sk * oi
    toi = ck * oi - sk * orr
    x1r = (er + tor) * inv
    x1i = (ei + toi) * inv
    x2r = (er - tor) * inv
    x2i = (ei - toi) * inv
    dr = dr_ref[...]
    di = di_ref[...]
    rb = rb_ref[...]
    ib = ib_ref[...]

    def freq_nl(xr, xi):
        yr = jax.nn.relu(xr * dr - xi * di + rb)
        yi = jax.nn.relu(xi * dr + xr * di + ib)
        return yr, yi

    x1r, x1i = freq_nl(x1r, x1i)
    x2r, x2i = freq_nl(x2r, x2i)
    u0r = x1r + x2r
    u0i = x1i + x2i
    d1r = x1r - x2r
    d1i = x1i - x2i
    v1r = d1r * ck - d1i * sk
    v1i = d1i * ck + d1r * sk
    ye = (_mm(C, u0r) - _mm(S, u0i)) * inv
    yo = (_mm(C, v1r) - _mm(S, v1i)) * inv
    o_ref[0, 0] = x_ref[0, 0] + _mm(ye, w2_ref[...]) + b2_ref[...]
    o_ref[0, 1] = x_ref[0, 1] + _mm(yo, w2_ref[...]) + b2_ref[...]


def _full(shape):
    return pl.BlockSpec(shape, lambda *a: tuple(0 for _ in shape))


def kernel(x, params):
    p = params
    Bb, C, Tt = x.shape
    n = Tt // RS
    dh = C * MLP
    rs1 = 1.0 / math.sqrt(1.0 + EPS)

    xts = x.transpose(0, 2, 1).reshape(Bb, n, RS, C).transpose(0, 2, 1, 3)
    xn1 = xts * (p['n1_g'] * rs1)[None, None, None, :] + p['n1_b']

    def conv_w(w):
        return w.transpose(2, 1, 0).reshape(3 * C, C)

    wq = conv_w(p['q_w'])
    wk = conv_w(p['k_w'])
    wv = p['v_w'][:, :, 0].T
    sq = (p['q_g'] * rs1)[None, :]
    bq = p['q_b'][None, :]
    sk = (p['k_g'] * rs1)[None, :]
    bk = p['k_b'][None, :]

    qp, kp, vp, q_r, k_r = pl.pallas_call(
        functools.partial(_qkv_kernel, C=C),
        grid=(Bb,),
        in_specs=[
            pl.BlockSpec((1, RS, n, C), lambda b: (b, 0, 0, 0)),
            _full((3 * C, C)), _full((3 * C, C)), _full((C, C)),
            _full((1, C)), _full((1, C)), _full((1, C)), _full((1, C)),
        ],
        out_specs=[
            pl.BlockSpec((1, RS, n, C), lambda b: (b, 0, 0, 0)),
            pl.BlockSpec((1, RS, n, C), lambda b: (b, 0, 0, 0)),
            pl.BlockSpec((1, RS, n, C), lambda b: (b, 0, 0, 0)),
            pl.BlockSpec((1, n, C), lambda b: (b, 0, 0)),
            pl.BlockSpec((1, n, C), lambda b: (b, 0, 0)),
        ],
        out_shape=[
            jax.ShapeDtypeStruct((Bb, RS, n, C), jnp.float32),
            jax.ShapeDtypeStruct((Bb, RS, n, C), jnp.float32),
            jax.ShapeDtypeStruct((Bb, RS, n, C), jnp.float32),
            jax.ShapeDtypeStruct((Bb, n, C), jnp.float32),
            jax.ShapeDtypeStruct((Bb, n, C), jnp.float32),
        ],
    )(xn1, wq, wk, wv, sq, bq, sk, bk)

    a_r = pl.pallas_call(
        _routing_kernel,
        grid=(Bb,),
        in_specs=[pl.BlockSpec((1, n, C), lambda b: (b, 0, 0)),
                  pl.BlockSpec((1, n, C), lambda b: (b, 0, 0))],
        out_specs=pl.BlockSpec((1, n, n), lambda b: (b, 0, 0)),
        out_shape=jax.ShapeDtypeStruct((Bb, n, n), jnp.float32),
    )(q_r, k_r)

    vals, _ = jax.lax.top_k(a_r, TOPK)
    thr = vals[:, :, TOPK - 1:TOPK]

    rb = 256
    wo = p['out_w'][:, :, 0].T
    ob = p['out_b'][None, :]
    lw = p['lepe_w'][:, 0, :].T
    lb = p['lepe_b'][None, :]
    scale = C ** (-0.5)

    y1 = pl.pallas_call(
        functools.partial(_attn_kernel, rb=rb, n=n, scale=scale),
        grid=(Bb, n // rb),
        in_specs=[
            pl.BlockSpec((1, RS, rb, C), lambda b, q: (b, 0, q, 0)),
            pl.BlockSpec((1, RS, n, C), lambda b, q: (b, 0, 0, 0)),
            pl.BlockSpec((1, RS, n, C), lambda b, q: (b, 0, 0, 0)),
            pl.BlockSpec((1, rb, n), lambda b, q: (b, q, 0)),
            pl.BlockSpec((1, rb, 1), lambda b, q: (b, q, 0)),
            pl.BlockSpec((1, RS, rb, C), lambda b, q: (b, 0, q, 0)),
            _full((C, C)), _full((1, C)), _full((3, C)), _full((1, C)),
        ],
        out_specs=pl.BlockSpec((1, RS, rb, C), lambda b, q: (b, 0, q, 0)),
        out_shape=jax.ShapeDtypeStruct((Bb, RS, n, C), jnp.float32),
    )(qp, kp, vp, a_r, thr, xts, wo, ob, lw, lb)

    w1 = p['fc1_w'][:, :, 0].T
    s1 = p['fc1_g'] * rs1
    w1f = (p['n2_g'] * rs1)[:, None] * w1 * s1[None, :]
    b1f = ((p['n2_b'] @ w1) * s1 + p['fc1_b'])[None, :]
    w2 = p['fc2_w'][:, :, 0].T
    w2f = w2 * (p['fc2_g'] * rs1)[None, :]
    b2f = p['fc2_b'][None, :]
    dr = jnp.diagonal(p['r'])[None, :]
    di = jnp.diagonal(p['i'])[None, :]
    rbv = p['rb'][None, :]
    ibv = p['ib'][None, :]
    kk = jnp.arange(n, dtype=jnp.float32)
    nnm = kk[:, None] * kk[None, :] * (2.0 * jnp.pi / n)
    Cm = jnp.cos(nnm)
    Sm = jnp.sin(nnm)
    ang = jnp.pi * kk / n
    tw = jnp.stack([jnp.cos(ang), jnp.sin(ang)], axis=1)

    out = pl.pallas_call(
        functools.partial(_ffn_kernel, T=Tt),
        grid=(Bb,),
        in_specs=[
            pl.BlockSpec((1, RS, n, C), lambda b: (b, 0, 0, 0)),
            _full((C, dh)), _full((1, dh)),
            _full((dh, C)), _full((1, C)),
            _full((1, dh)), _full((1, dh)), _full((1, dh)), _full((1, dh)),
            _full((n, n)), _full((n, n)), _full((n, 2)),
        ],
        out_specs=pl.BlockSpec((1, RS, n, C), lambda b: (b, 0, 0, 0)),
        out_shape=jax.ShapeDtypeStruct((Bb, RS, n, C), jnp.float32),
    )(y1, w1f, b1f, w2f, b2f, dr, di, rbv, ibv, Cm, Sm, tw)

    return out.transpose(0, 3, 2, 1).reshape(Bb, C, Tt)


# speedup vs baseline: 4.0054x; 2.3821x over previous
import functools
import math

import jax
import jax.numpy as jnp
from jax.experimental import pallas as pl

DIM = 256
HEADS = 4
TOPK = 40
MLP = 2
EPS = 1e-5
RS = 2

HD = DIM // HEADS
NEG = -1e30


def _mm(a, b, dims=((1,), (0,))):
    return jax.lax.dot_general(a, b, (dims, ((), ())),
                               preferred_element_type=jnp.float32)


def _shift_down(a):
    return jnp.concatenate([jnp.zeros((1, a.shape[1]), a.dtype), a[:-1]], 0)


def _shift_up(a):
    return jnp.concatenate([a[1:], jnp.zeros((1, a.shape[1]), a.dtype)], 0)


def _qkv_kernel(x_ref, wq_ref, wk_ref, wv_ref, sq_ref, bq_ref, sk_ref,
                bk_ref, qp_ref, kp_ref, vp_ref, qr_ref, kr_ref, *, C):
    xe = x_ref[0, 0]
    xo = x_ref[0, 1]
    xo_m = _shift_down(xo)
    xe_p = _shift_up(xe)

    def conv3(w_ref):
        w0 = w_ref[0:C]
        w1 = w_ref[C:2 * C]
        w2 = w_ref[2 * C:3 * C]
        y_e = _mm(xo_m, w0) + _mm(xe, w1) + _mm(xo, w2)
        y_o = _mm(xe, w0) + _mm(xo, w1) + _mm(xe_p, w2)
        return y_e, y_o

    q_e, q_o = conv3(wq_ref)
    q_e = q_e * sq_ref[...] + bq_ref[...]
    q_o = q_o * sq_ref[...] + bq_ref[...]
    k_e, k_o = conv3(wk_ref)
    k_e = k_e * sk_ref[...] + bk_ref[...]
    k_o = k_o * sk_ref[...] + bk_ref[...]
    wv = wv_ref[...]
    v_e = _mm(xe, wv)
    v_o = _mm(xo, wv)
    qp_ref[0, 0] = q_e
    qp_ref[0, 1] = q_o
    kp_ref[0, 0] = k_e
    kp_ref[0, 1] = k_o
    vp_ref[0, 0] = v_e
    vp_ref[0, 1] = v_o
    qr_ref[0] = (q_e + q_o) * 0.5
    kr_ref[0] = (k_e + k_o) * 0.5


def _routing_kernel(qr_ref, kr_ref, ar_ref, thr_ref):
    a = _mm(qr_ref[0], kr_ref[0], dims=((1,), (1,)))
    ar_ref[0] = a
    n = a.shape[0]
    bits = jax.lax.bitcast_convert_type(a, jnp.int32)
    imin = jnp.int32(-2147483648)
    u = jnp.where(bits < 0, imin - bits, bits)
    lo0 = jnp.full((n, 1), imin, jnp.int32)
    hi0 = jnp.full((n, 1), 2147483647, jnp.int32)

    def body(i, lh):
        lo, hi = lh
        x = lo ^ hi
        mid = (lo & hi) + (x >> 1) + (x & 1)
        cnt = jnp.sum((u >= mid).astype(jnp.int32), axis=1, keepdims=True)
        ge = cnt >= TOPK
        return jnp.where(ge, mid, lo), jnp.where(ge, hi, mid - 1)

    lo, _ = jax.lax.fori_loop(0, 32, body, (lo0, hi0))
    tb = jnp.where(lo > 0, lo, imin - lo)
    thr_ref[0] = jax.lax.bitcast_convert_type(tb, jnp.float32)


def _attn_kernel(q_ref, k_ref, v_ref, ar_ref, t_ref, x_ref, wo_ref, ob_ref,
                 lw_ref, lb_ref, o_ref, *, rb, n, scale):
    qb = pl.program_id(1)
    mask = ar_ref[0] >= t_ref[0]
    ve = v_ref[0, 0]
    vo = v_ref[0, 1]
    outs = []
    for i in range(RS):
        q_i = q_ref[0, i]
        heads = []
        for h in range(HEADS):
            sl = slice(h * HD, (h + 1) * HD)
            qh = q_i[:, sl]
            s0 = _mm(qh, k_ref[0, 0][:, sl], dims=((1,), (1,)))
            s1 = _mm(qh, k_ref[0, 1][:, sl], dims=((1,), (1,)))
            s0 = jnp.where(mask, s0 * scale, NEG)
            s1 = jnp.where(mask, s1 * scale, NEG)
            mx = jnp.maximum(jnp.max(s0, 1, keepdims=True),
                             jnp.max(s1, 1, keepdims=True))
            p0 = jnp.exp(s0 - mx)
            p1 = jnp.exp(s1 - mx)
            den = (jnp.sum(p0, 1, keepdims=True)
                   + jnp.sum(p1, 1, keepdims=True))
            oh = _mm(p0, ve[:, sl]) + _mm(p1, vo[:, sl])
            heads.append(oh / den)
        outs.append(jnp.concatenate(heads, axis=1))

    r0 = qb * rb
    nq = pl.num_programs(1)
    ve_b = v_ref[0, 0, pl.ds(r0, rb), :]
    vo_b = v_ref[0, 1, pl.ds(r0, rb), :]
    prev = v_ref[0, 1, pl.ds(jnp.maximum(r0 - 1, 0), 1), :]
    prev = jnp.where(qb > 0, prev, 0.0)
    nxt = v_ref[0, 0, pl.ds(jnp.minimum(r0 + rb, n - 1), 1), :]
    nxt = jnp.where(qb < nq - 1, nxt, 0.0)
    vom_b = jnp.concatenate([prev, vo_b[:-1]], 0)
    vep_b = jnp.concatenate([ve_b[1:], nxt], 0)
    lw0 = lw_ref[0:1]
    lw1 = lw_ref[1:2]
    lw2 = lw_ref[2:3]
    lb = lb_ref[...]
    lepe_e = lw0 * vom_b + lw1 * ve_b + lw2 * vo_b + lb
    lepe_o = lw0 * ve_b + lw1 * vo_b + lw2 * vep_b + lb
    wo = wo_ref[...]
    ob = ob_ref[...]
    o_ref[0, 0] = x_ref[0, 0] + _mm(outs[0] + lepe_e, wo) + ob
    o_ref[0, 1] = x_ref[0, 1] + _mm(outs[1] + lepe_o, wo) + ob


def _ffn_kernel(x_ref, w1_ref, b1_ref, w2_ref, b2_ref, dr_ref, di_ref,
                rb_ref, ib_ref, c_ref, s_ref, tw_ref, o_ref, *, T):
    n = T // 2
    inv = 1.0 / math.sqrt(float(T))
    e = jax.nn.relu(_mm(x_ref[0, 0], w1_ref[...]) + b1_ref[...])
    o = jax.nn.relu(_mm(x_ref[0, 1], w1_ref[...]) + b1_ref[...])
    C = c_ref[...]
    S = s_ref[...]
    er = _mm(C, e)
    ei = -_mm(S, e)
    orr = _mm(C, o)
    oi = -_mm(S, o)
    ck = tw_ref[:, 0:1]
    sk = tw_ref[:, 1:2]
    tor = ck * orr + sk * oi
    toi = ck * oi - sk * orr
    x1r = (er + tor) * inv
    x1i = (ei + toi) * inv
    x2r = (er - tor) * inv
    x2i = (ei - toi) * inv
    dr = dr_ref[...]
    di = di_ref[...]
    rb = rb_ref[...]
    ib = ib_ref[...]

    def freq_nl(xr, xi):
        yr = jax.nn.relu(xr * dr - xi * di + rb)
        yi = jax.nn.relu(xi * dr + xr * di + ib)
        return yr, yi

    x1r, x1i = freq_nl(x1r, x1i)
    x2r, x2i = freq_nl(x2r, x2i)
    u0r = x1r + x2r
    u0i = x1i + x2i
    d1r = x1r - x2r
    d1i = x1i - x2i
    v1r = d1r * ck - d1i * sk
    v1i = d1i * ck + d1r * sk
    ye = (_mm(C, u0r) - _mm(S, u0i)) * inv
    yo = (_mm(C, v1r) - _mm(S, v1i)) * inv
    o_ref[0, 0] = x_ref[0, 0] + _mm(ye, w2_ref[...]) + b2_ref[...]
    o_ref[0, 1] = x_ref[0, 1] + _mm(yo, w2_ref[...]) + b2_ref[...]


def _full(shape):
    return pl.BlockSpec(shape, lambda *a: tuple(0 for _ in shape))


def kernel(x, params):
    p = params
    Bb, C, Tt = x.shape
    n = Tt // RS
    dh = C * MLP
    rs1 = 1.0 / math.sqrt(1.0 + EPS)

    xts = x.transpose(0, 2, 1).reshape(Bb, n, RS, C).transpose(0, 2, 1, 3)
    xn1 = xts * (p['n1_g'] * rs1)[None, None, None, :] + p['n1_b']

    def conv_w(w):
        return w.transpose(2, 1, 0).reshape(3 * C, C)

    wq = conv_w(p['q_w'])
    wk = conv_w(p['k_w'])
    wv = p['v_w'][:, :, 0].T
    sq = (p['q_g'] * rs1)[None, :]
    bq = p['q_b'][None, :]
    sk = (p['k_g'] * rs1)[None, :]
    bk = p['k_b'][None, :]

    qp, kp, vp, q_r, k_r = pl.pallas_call(
        functools.partial(_qkv_kernel, C=C),
        grid=(Bb,),
        in_specs=[
            pl.BlockSpec((1, RS, n, C), lambda b: (b, 0, 0, 0)),
            _full((3 * C, C)), _full((3 * C, C)), _full((C, C)),
            _full((1, C)), _full((1, C)), _full((1, C)), _full((1, C)),
        ],
        out_specs=[
            pl.BlockSpec((1, RS, n, C), lambda b: (b, 0, 0, 0)),
            pl.BlockSpec((1, RS, n, C), lambda b: (b, 0, 0, 0)),
            pl.BlockSpec((1, RS, n, C), lambda b: (b, 0, 0, 0)),
            pl.BlockSpec((1, n, C), lambda b: (b, 0, 0)),
            pl.BlockSpec((1, n, C), lambda b: (b, 0, 0)),
        ],
        out_shape=[
            jax.ShapeDtypeStruct((Bb, RS, n, C), jnp.float32),
            jax.ShapeDtypeStruct((Bb, RS, n, C), jnp.float32),
            jax.ShapeDtypeStruct((Bb, RS, n, C), jnp.float32),
            jax.ShapeDtypeStruct((Bb, n, C), jnp.float32),
            jax.ShapeDtypeStruct((Bb, n, C), jnp.float32),
        ],
    )(xn1, wq, wk, wv, sq, bq, sk, bk)

    a_r, thr = pl.pallas_call(
        _routing_kernel,
        grid=(Bb,),
        in_specs=[pl.BlockSpec((1, n, C), lambda b: (b, 0, 0)),
                  pl.BlockSpec((1, n, C), lambda b: (b, 0, 0))],
        out_specs=[pl.BlockSpec((1, n, n), lambda b: (b, 0, 0)),
                   pl.BlockSpec((1, n, 1), lambda b: (b, 0, 0))],
        out_shape=[jax.ShapeDtypeStruct((Bb, n, n), jnp.float32),
                   jax.ShapeDtypeStruct((Bb, n, 1), jnp.float32)],
    )(q_r, k_r)

    rb = 256
    wo = p['out_w'][:, :, 0].T
    ob = p['out_b'][None, :]
    lw = p['lepe_w'][:, 0, :].T
    lb = p['lepe_b'][None, :]
    scale = C ** (-0.5)

    y1 = pl.pallas_call(
        functools.partial(_attn_kernel, rb=rb, n=n, scale=scale),
        grid=(Bb, n // rb),
        in_specs=[
            pl.BlockSpec((1, RS, rb, C), lambda b, q: (b, 0, q, 0)),
            pl.BlockSpec((1, RS, n, C), lambda b, q: (b, 0, 0, 0)),
            pl.BlockSpec((1, RS, n, C), lambda b, q: (b, 0, 0, 0)),
            pl.BlockSpec((1, rb, n), lambda b, q: (b, q, 0)),
            pl.BlockSpec((1, rb, 1), lambda b, q: (b, q, 0)),
            pl.BlockSpec((1, RS, rb, C), lambda b, q: (b, 0, q, 0)),
            _full((C, C)), _full((1, C)), _full((3, C)), _full((1, C)),
        ],
        out_specs=pl.BlockSpec((1, RS, rb, C), lambda b, q: (b, 0, q, 0)),
        out_shape=jax.ShapeDtypeStruct((Bb, RS, n, C), jnp.float32),
    )(qp, kp, vp, a_r, thr, xts, wo, ob, lw, lb)

    w1 = p['fc1_w'][:, :, 0].T
    s1 = p['fc1_g'] * rs1
    w1f = (p['n2_g'] * rs1)[:, None] * w1 * s1[None, :]
    b1f = ((p['n2_b'] @ w1) * s1 + p['fc1_b'])[None, :]
    w2 = p['fc2_w'][:, :, 0].T
    w2f = w2 * (p['fc2_g'] * rs1)[None, :]
    b2f = p['fc2_b'][None, :]
    dr = jnp.diagonal(p['r'])[None, :]
    di = jnp.diagonal(p['i'])[None, :]
    rbv = p['rb'][None, :]
    ibv = p['ib'][None, :]
    kk = jnp.arange(n, dtype=jnp.float32)
    nnm = kk[:, None] * kk[None, :] * (2.0 * jnp.pi / n)
    Cm = jnp.cos(nnm)
    Sm = jnp.sin(nnm)
    ang = jnp.pi * kk / n
    tw = jnp.stack([jnp.cos(ang), jnp.sin(ang)], axis=1)

    out = pl.pallas_call(
        functools.partial(_ffn_kernel, T=Tt),
        grid=(Bb,),
        in_specs=[
            pl.BlockSpec((1, RS, n, C), lambda b: (b, 0, 0, 0)),
            _full((C, dh)), _full((1, dh)),
            _full((dh, C)), _full((1, C)),
            _full((1, dh)), _full((1, dh)), _full((1, dh)), _full((1, dh)),
            _full((n, n)), _full((n, n)), _full((n, 2)),
        ],
        out_specs=pl.BlockSpec((1, RS, n, C), lambda b: (b, 0, 0, 0)),
        out_shape=jax.ShapeDtypeStruct((Bb, RS, n, C), jnp.float32),
    )(y1, w1f, b1f, w2f, b2f, dr, di, rbv, ibv, Cm, Sm, tw)

    return out.transpose(0, 3, 2, 1).reshape(Bb, C, Tt)
